# baseline (speedup 1.0000x reference)
import jax
import jax.numpy as jnp
from jax import lax
from jax.experimental import pallas as pl
from jax.experimental.pallas import tpu as pltpu

_BM = 512
_NK = 4
_BK = 2560
_NPAD = 10240


def _gcn_body(input_ref, weight_ref, adj_ref, out_ref, support_ref):
    i = pl.program_id(0)
    k = pl.program_id(1)

    @pl.when((i == 0) & (k == 0))
    def _compute_support():
        x = input_ref[...].astype(jnp.bfloat16)
        w = weight_ref[...].astype(jnp.bfloat16)
        s = jnp.dot(x, w, preferred_element_type=jnp.float32)
        support_ref[: x.shape[0], :] = s.astype(jnp.bfloat16)
        support_ref[pl.ds(x.shape[0], _NPAD - x.shape[0]), :] = jnp.zeros(
            (_NPAD - x.shape[0], s.shape[1]), jnp.bfloat16)

    cols = k * _BK + lax.broadcasted_iota(jnp.int32, (1, _BK), 1)
    a = jnp.where(cols < 10000, adj_ref[...], 0.0).astype(jnp.bfloat16)
    s = support_ref[pl.ds(k * _BK, _BK), :]
    part = jnp.dot(a, s, preferred_element_type=jnp.float32)

    @pl.when(k == 0)
    def _init():
        out_ref[...] = part

    @pl.when(k > 0)
    def _accum():
        out_ref[...] += part

    @pl.when(k == _NK - 1)
    def _finish():
        out_ref[...] = jnp.maximum(out_ref[...], 0.0)


def kernel(input, adj, weight):
    n, d_in = input.shape
    d_out = weight.shape[1]
    return pl.pallas_call(
        _gcn_body,
        grid=(pl.cdiv(n, _BM), _NK),
        in_specs=[
            pl.BlockSpec((n, d_in), lambda i, k: (0, 0)),
            pl.BlockSpec((d_in, d_out), lambda i, k: (0, 0)),
            pl.BlockSpec((_BM, _BK), lambda i, k: (i, k)),
        ],
        out_specs=pl.BlockSpec((_BM, d_out), lambda i, k: (i, 0)),
        out_shape=jax.ShapeDtypeStruct((n, d_out), jnp.float32),
        scratch_shapes=[pltpu.VMEM((_NPAD, d_out), jnp.bfloat16)],
    )(input.astype(jnp.float32), weight, adj)


# speedup vs baseline: 1.1675x; 1.1675x over previous
import jax
import jax.numpy as jnp
from jax.experimental import pallas as pl
from jax.experimental.pallas import tpu as pltpu

_BM = 512


def _gcn_body(input_ref, weight_ref, adj_ref, out_ref, support_ref):
    @pl.when(pl.program_id(0) == 0)
    def _compute_support():
        x = input_ref[...].astype(jnp.bfloat16)
        w = weight_ref[...].astype(jnp.bfloat16)
        s = jnp.dot(x, w, preferred_element_type=jnp.float32)
        support_ref[...] = s.astype(jnp.bfloat16)

    a = adj_ref[...].astype(jnp.bfloat16)
    acc = jnp.dot(a, support_ref[...], preferred_element_type=jnp.float32)
    out_ref[...] = jnp.maximum(acc, 0.0)


def kernel(input, adj, weight):
    n, d_in = input.shape
    d_out = weight.shape[1]
    return pl.pallas_call(
        _gcn_body,
        grid=(pl.cdiv(n, _BM),),
        in_specs=[
            pl.BlockSpec((n, d_in), lambda i: (0, 0)),
            pl.BlockSpec((d_in, d_out), lambda i: (0, 0)),
            pl.BlockSpec((_BM, n), lambda i: (i, 0)),
        ],
        out_specs=pl.BlockSpec((_BM, d_out), lambda i: (i, 0)),
        out_shape=jax.ShapeDtypeStruct((n, d_out), jnp.float32),
        scratch_shapes=[pltpu.VMEM((n, d_out), jnp.bfloat16)],
    )(input.astype(jnp.float32), weight, adj)
